# baseline (speedup 1.0000x reference)
import functools

import jax
import jax.numpy as jnp
from jax import lax
from jax.experimental import pallas as pl
from jax.experimental.pallas import tpu as pltpu

F32 = jnp.float32
_EPS = 1e-5
_K = 32
_N = 4096
_B = 16


def _k1_body(ptsP_ref, centers_ref, mask_ref):
    d2 = jnp.zeros((32, _N), F32)
    for k in range(3):
        ck = centers_ref[:, k:k + 1]
        diff = ptsP_ref[k] - ck
        d2 = d2 + diff * diff
    d = jnp.sqrt(d2)
    bits = lax.bitcast_convert_type(d, jnp.int32)

    def bs_body(_, carry):
        lo, hi = carry
        mid = lo + ((hi - lo) >> 1)
        cnt = jnp.sum((bits <= mid).astype(jnp.int32), axis=1, keepdims=True)
        ge = cnt >= _K
        return jnp.where(ge, lo, mid + 1), jnp.where(ge, mid, hi)

    lo0 = jnp.zeros((32, 1), jnp.int32)
    hi0 = jnp.full((32, 1), 0x7F800000, jnp.int32)
    lo, hi = lax.fori_loop(0, 31, bs_body, (lo0, hi0))
    t = lo

    lt = bits < t
    eq = bits == t
    cnt_lt = jnp.sum(lt.astype(jnp.int32), axis=1, keepdims=True)
    m = _K - cnt_lt
    idx = lax.broadcasted_iota(jnp.int32, (32, _N), 1)

    def ibs_body(_, carry):
        lo, hi = carry
        mid = lo + ((hi - lo) >> 1)
        cnt = jnp.sum((eq & (idx <= mid)).astype(jnp.int32), axis=1, keepdims=True)
        ge = cnt >= m
        return jnp.where(ge, lo, mid + 1), jnp.where(ge, mid, hi)

    ilo, ihi = lax.fori_loop(0, 12, ibs_body,
                             (jnp.zeros((32, 1), jnp.int32),
                              jnp.full((32, 1), _N - 1, jnp.int32)))
    sel = lt | (eq & (idx <= ilo))
    mask_ref[...] = jnp.where(sel, 1.0, 0.0).astype(F32)


def _select_masks(ptsP, centers):
    return pl.pallas_call(
        _k1_body,
        out_shape=jax.ShapeDtypeStruct((32, _N), F32),
    )(ptsP, centers)


def _gn_scale_shift(s, q, gamma, beta, groups, n_per_group):
    C = s.shape[0]
    cpg = C // groups
    sg = jnp.sum(s.reshape(groups, cpg, 1), axis=1, keepdims=True)
    qg = jnp.sum(q.reshape(groups, cpg, 1), axis=1, keepdims=True)
    mean = sg / n_per_group
    var = qg / n_per_group - mean * mean
    inv = lax.rsqrt(var + _EPS)
    inv_c = jnp.broadcast_to(inv, (groups, cpg, 1)).reshape(C, 1)
    mean_c = jnp.broadcast_to(mean, (groups, cpg, 1)).reshape(C, 1)
    scale = inv_c * gamma
    shift = beta - mean_c * scale
    return scale, shift


def _gn_relu(y, gamma, beta, groups):
    C = y.shape[0]
    n = y.shape[1] * (C // groups)
    s = jnp.sum(y, axis=1, keepdims=True)
    q = jnp.sum(y * y, axis=1, keepdims=True)
    scale, shift = _gn_scale_shift(s, q, gamma, beta, groups, n)
    return jnp.maximum(y * scale + shift, 0.0)


def _k2_body(xT_ref, mask_ref,
             w1_ref, b1_ref, g1_ref, be1_ref,
             w2_ref, b2_ref, g2_ref, be2_ref,
             w3f_ref, b3_ref, flip_ref, g3_ref, be3_ref,
             gmax_ref, jm_ref, dm_ref):
    BF = jnp.bfloat16
    x = xT_ref[0]
    y1 = jnp.dot(w1_ref[...], x, preferred_element_type=F32) + b1_ref[...]
    h1 = _gn_relu(y1, g1_ref[...], be1_ref[...], 16)
    y2 = jnp.dot(w2_ref[...].astype(BF), h1.astype(BF),
                 preferred_element_type=F32) + b2_ref[...]
    h2 = _gn_relu(y2, g2_ref[...], be2_ref[...], 32)
    h2b = h2.astype(BF)
    w3b = w3f_ref[...].astype(BF)

    rsum = jnp.sum(h2, axis=1, keepdims=True)
    s2 = jnp.dot(w3f_ref[...], rsum, preferred_element_type=F32)
    G = lax.dot_general(h2b, h2b, (((1,), (1,)), ((), ())),
                        preferred_element_type=F32)
    M = jnp.dot(w3f_ref[...], G, preferred_element_type=F32)
    q2 = jnp.sum(M * w3f_ref[...], axis=1, keepdims=True)

    NT = 4
    TN = _N // NT
    neg = jnp.float32(-jnp.inf)
    mg = jnp.full((1024, 1), neg, F32)
    mj = jnp.full((1024, 1), neg, F32)
    md = jnp.full((1024, 1), neg, F32)
    mrow = mask_ref[0]
    for t in range(NT):
        sl = slice(t * TN, (t + 1) * TN)
        raw = jnp.dot(w3b, h2b[:, sl], preferred_element_type=F32)
        mg = jnp.maximum(mg, jnp.max(raw, axis=1, keepdims=True))
        selj = jnp.where(mrow[0:1, sl] > 0, raw, neg)
        mj = jnp.maximum(mj, jnp.max(selj, axis=1, keepdims=True))
        seld = jnp.where(mrow[1:2, sl] > 0, raw, neg)
        md = jnp.maximum(md, jnp.max(seld, axis=1, keepdims=True))

    flip = flip_ref[...]
    b3 = b3_ref[...]
    n = jnp.float32(_N)
    s = flip * s2 + n * b3
    q = q2 + 2.0 * b3 * flip * s2 + n * b3 * b3
    scale, shift = _gn_scale_shift(s, q, g3_ref[...], be3_ref[...], 64, 16 * _N)
    w = scale * flip
    fb3 = flip * b3
    gmax_ref[...] = jnp.maximum(w * (mg + fb3) + shift, 0.0).reshape(1, 1024, 1)
    jm_ref[...] = jnp.maximum(w * (mj + fb3) + shift, 0.0).reshape(1, 1024, 1)
    dm_ref[...] = jnp.maximum(w * (md + fb3) + shift, 0.0).reshape(1, 1024, 1)


def _encode(xT, masks, w1, b1, g1, be1, w2, b2, g2, be2,
            w3f, b3, flip, g3, be3):
    col = jax.ShapeDtypeStruct((_B, 1024, 1), F32)
    full = lambda shape: pl.BlockSpec(shape, lambda b: (0,) * len(shape))
    return pl.pallas_call(
        _k2_body,
        grid=(_B,),
        in_specs=[
            pl.BlockSpec((1, 4, _N), lambda b: (b, 0, 0)),
            pl.BlockSpec((1, 2, _N), lambda b: (b, 0, 0)),
            full((128, 4)), full((128, 1)), full((128, 1)), full((128, 1)),
            full((256, 128)), full((256, 1)), full((256, 1)), full((256, 1)),
            full((1024, 256)), full((1024, 1)), full((1024, 1)),
            full((1024, 1)), full((1024, 1)),
        ],
        out_specs=[
            pl.BlockSpec((1, 1024, 1), lambda b: (b, 0, 0)),
            pl.BlockSpec((1, 1024, 1), lambda b: (b, 0, 0)),
            pl.BlockSpec((1, 1024, 1), lambda b: (b, 0, 0)),
        ],
        out_shape=[col, col, col],
    )(xT, masks, w1, b1, g1, be1, w2, b2, g2, be2, w3f, b3, flip, g3, be3)


def _linT(x, w, b):
    return lax.dot_general(x, w, (((1,), (1,)), ((), ())),
                           preferred_element_type=F32) + b


def _ln(x, g, b):
    m = jnp.mean(x, axis=-1, keepdims=True)
    v = jnp.mean(x * x, axis=-1, keepdims=True) - m * m
    return (x - m) * lax.rsqrt(v + _EPS) * g + b


def _relu(x):
    return jnp.maximum(x, 0.0)


def _k3_body(g_ref, jm_ref, dm_ref, dp_ref, dv_ref, jt_ref, ja_ref, jo_ref,
             *prefs):
    names = _K3_PARAM_NAMES
    p = {nm: r[...] for nm, r in zip(names, prefs)}
    g0 = g_ref[...]
    jm_in = jm_ref[...]
    dm_in = dm_ref[...]
    dp = dp_ref[...]
    dv = dv_ref[...]
    jt = jt_ref[...]
    ja = ja_ref[...]
    jo = jo_ref[...]

    g = _relu(_ln(_linT(g0, p['pc_w4'], p['pc_b4']), p['pc_ln4g'], p['pc_ln4b']))
    g = _linT(g, p['pc_w5'], p['pc_b5'])

    di = jnp.concatenate([dp, dv], axis=1)
    direction_feat = _linT(_ln(_relu(_linT(di, p['de_w1'], p['de_b1'])),
                               p['de_lng'], p['de_lnb']), p['de_w2'], p['de_b2'])
    rel = dp - jo
    relf = _linT(_ln(_relu(_linT(rel, p['rp_w1'], p['rp_b1'])),
                     p['rp_lng'], p['rp_lnb']), p['rp_w2'], p['rp_b2'])
    mag = jnp.sqrt(jnp.sum(dv * dv, axis=1, keepdims=True))
    magf = _linT(_relu(_linT(mag, p['mg_w1'], p['mg_b1'])), p['mg_w2'], p['mg_b2'])
    comb = jnp.concatenate([direction_feat, relf, magf], axis=1)
    drag_feat = _linT(_relu(_linT(comb, p['df_w1'], p['df_b1'])),
                      p['df_w2'], p['df_b2'])

    tf = jnp.where(jt == 0, p['emb'][0:1, :], p['emb'][1:2, :])
    af = _linT(_relu(_linT(ja, p['ax_w1'], p['ax_b1'])), p['ax_w2'], p['ax_b2'])
    of = _linT(_relu(_linT(jo, p['or_w1'], p['or_b1'])), p['or_w2'], p['or_b2'])
    jc = jnp.concatenate([tf, af, of], axis=1)
    joint_feat = _linT(_relu(_linT(jc, p['jf_w1'], p['jf_b1'])),
                       p['jf_w2'], p['jf_b2'])

    jl = _linT(_relu(_linT(jm_in, p['jm_w1'], p['jm_b1'])), p['jm_w2'], p['jm_b2'])
    dl = _linT(_relu(_linT(dm_in, p['dm_w1'], p['dm_b1'])), p['dm_w2'], p['dm_b2'])
    local = _linT(_relu(_linT(jnp.concatenate([jl, dl], axis=1),
                              p['lf_w1'], p['lf_b1'])), p['lf_w2'], p['lf_b2'])
    local = _linT(joint_feat, p['fs_w'], p['fs_b']) * local + \
        _linT(joint_feat, p['fsh_w'], p['fsh_b'])

    vi = jnp.concatenate([local, joint_feat, drag_feat], axis=1)
    mu = _linT(vi, p['mu_w'], p['mu_b'])
    lv = _linT(vi, p['lv_w'], p['lv_b'])
    out_ref = prefs[len(names)]
    out_ref[...] = jnp.concatenate([mu, lv, g], axis=1)


_K3_PARAM_NAMES = (
    'pc_w4', 'pc_b4', 'pc_ln4g', 'pc_ln4b', 'pc_w5', 'pc_b5',
    'de_w1', 'de_b1', 'de_lng', 'de_lnb', 'de_w2', 'de_b2',
    'rp_w1', 'rp_b1', 'rp_lng', 'rp_lnb', 'rp_w2', 'rp_b2',
    'mg_w1', 'mg_b1', 'mg_w2', 'mg_b2',
    'df_w1', 'df_b1', 'df_w2', 'df_b2',
    'emb',
    'ax_w1', 'ax_b1', 'ax_w2', 'ax_b2',
    'or_w1', 'or_b1', 'or_w2', 'or_b2',
    'jf_w1', 'jf_b1', 'jf_w2', 'jf_b2',
    'jm_w1', 'jm_b1', 'jm_w2', 'jm_b2',
    'dm_w1', 'dm_b1', 'dm_w2', 'dm_b2',
    'lf_w1', 'lf_b1', 'lf_w2', 'lf_b2',
    'fs_w', 'fs_b', 'fsh_w', 'fsh_b',
    'mu_w', 'mu_b', 'lv_w', 'lv_b',
)

def _k3_entry(*refs):
    _k3_body(*refs)


def _head(gT, jmT, dmT, dp, dv, jt2, ja, jo, plist):
    return pl.pallas_call(
        _k3_entry,
        out_shape=jax.ShapeDtypeStruct((_B, 2048), F32),
    )(gT, jmT, dmT, dp, dv, jt2, ja, jo, *plist)


def kernel(points, drag_point, drag_vector, joint_type, joint_axis,
           joint_origin, params):
    p = params
    pts = points[..., :3]
    xT = jnp.transpose(points, (0, 2, 1))
    ptsT = jnp.transpose(pts, (0, 2, 1))
    ptsP = jnp.transpose(jnp.repeat(ptsT, 2, axis=0), (1, 0, 2))
    centers = jnp.stack([joint_origin, drag_point], axis=1).reshape(32, 3)

    masks32 = _select_masks(ptsP, centers)
    masks = masks32.reshape(_B, 2, _N)

    c1 = lambda a: a.reshape(-1, 1)
    flip = jnp.where(p['pc_g3'] >= 0, 1.0, -1.0).astype(F32)
    w3f = p['pc_w3'] * flip[:, None]
    gcols, jmcols, dmcols = _encode(
        xT, masks,
        p['pc_w1'], c1(p['pc_b1']), c1(p['pc_g1']), c1(p['pc_be1']),
        p['pc_w2'], c1(p['pc_b2']), c1(p['pc_g2']), c1(p['pc_be2']),
        w3f, c1(p['pc_b3']), c1(flip), c1(p['pc_g3']), c1(p['pc_be3']))

    r1 = lambda a: a.reshape(1, -1)
    plist = []
    for nm in _K3_PARAM_NAMES:
        a = p[nm]
        plist.append(r1(a) if a.ndim == 1 else a)
    gcols = gcols.reshape(_B, 1024)
    jmcols = jmcols.reshape(_B, 1024)
    dmcols = dmcols.reshape(_B, 1024)
    return _head(gcols, jmcols, dmcols,
                 drag_point, drag_vector,
                 joint_type.astype(jnp.int32).reshape(_B, 1),
                 joint_axis, joint_origin, plist)


# speedup vs baseline: 1.0310x; 1.0310x over previous
import functools

import jax
import jax.numpy as jnp
from jax import lax
from jax.experimental import pallas as pl
from jax.experimental.pallas import tpu as pltpu

F32 = jnp.float32
_EPS = 1e-5
_K = 32
_N = 4096
_B = 16


def _k1_body(xT_ref, jo_ref, dp_ref, maskj_ref, maskd_ref):
    dj = jnp.zeros((_B, _N), F32)
    dd = jnp.zeros((_B, _N), F32)
    for k in range(3):
        col = xT_ref[:, k, :]
        fj = col - jo_ref[:, k:k + 1]
        dj = dj + fj * fj
        fd = col - dp_ref[:, k:k + 1]
        dd = dd + fd * fd
    d = jnp.sqrt(jnp.concatenate([dj, dd], axis=0))
    bits = lax.bitcast_convert_type(d, jnp.int32)

    def bs_body(_, carry):
        lo, hi = carry
        mid = lo + ((hi - lo) >> 1)
        cnt = jnp.sum((bits <= mid).astype(jnp.int32), axis=1, keepdims=True)
        ge = cnt >= _K
        return jnp.where(ge, lo, mid + 1), jnp.where(ge, mid, hi)

    lo0 = jnp.zeros((32, 1), jnp.int32)
    hi0 = jnp.full((32, 1), 0x7F800000, jnp.int32)
    lo, hi = lax.fori_loop(0, 31, bs_body, (lo0, hi0))
    t = lo

    lt = bits < t
    eq = bits == t
    cnt_lt = jnp.sum(lt.astype(jnp.int32), axis=1, keepdims=True)
    m = _K - cnt_lt
    idx = lax.broadcasted_iota(jnp.int32, (32, _N), 1)

    def ibs_body(_, carry):
        lo, hi = carry
        mid = lo + ((hi - lo) >> 1)
        cnt = jnp.sum((eq & (idx <= mid)).astype(jnp.int32), axis=1, keepdims=True)
        ge = cnt >= m
        return jnp.where(ge, lo, mid + 1), jnp.where(ge, mid, hi)

    ilo, ihi = lax.fori_loop(0, 12, ibs_body,
                             (jnp.zeros((32, 1), jnp.int32),
                              jnp.full((32, 1), _N - 1, jnp.int32)))
    sel = lt | (eq & (idx <= ilo))
    m = jnp.where(sel, 1.0, 0.0).astype(F32)
    maskj_ref[...] = m[:_B].reshape(_B, 1, _N)
    maskd_ref[...] = m[_B:].reshape(_B, 1, _N)


def _select_masks(xT, jo, dp):
    out = jax.ShapeDtypeStruct((_B, 1, _N), F32)
    return pl.pallas_call(
        _k1_body,
        out_shape=[out, out],
    )(xT, jo, dp)


def _gn_scale_shift(s, q, gamma, beta, groups, n_per_group):
    C = s.shape[0]
    cpg = C // groups
    sg = jnp.sum(s.reshape(groups, cpg, 1), axis=1, keepdims=True)
    qg = jnp.sum(q.reshape(groups, cpg, 1), axis=1, keepdims=True)
    mean = sg / n_per_group
    var = qg / n_per_group - mean * mean
    inv = lax.rsqrt(var + _EPS)
    inv_c = jnp.broadcast_to(inv, (groups, cpg, 1)).reshape(C, 1)
    mean_c = jnp.broadcast_to(mean, (groups, cpg, 1)).reshape(C, 1)
    scale = inv_c * gamma
    shift = beta - mean_c * scale
    return scale, shift


def _gn_relu(y, gamma, beta, groups):
    C = y.shape[0]
    n = y.shape[1] * (C // groups)
    s = jnp.sum(y, axis=1, keepdims=True)
    q = jnp.sum(y * y, axis=1, keepdims=True)
    scale, shift = _gn_scale_shift(s, q, gamma, beta, groups, n)
    return jnp.maximum(y * scale + shift, 0.0)


def _k2_body(xT_ref, maskj_ref, maskd_ref,
             w1_ref, b1_ref, g1_ref, be1_ref,
             w2_ref, b2_ref, g2_ref, be2_ref,
             w3f_ref, b3_ref, flip_ref, g3_ref, be3_ref,
             gmax_ref, jm_ref, dm_ref):
    BF = jnp.bfloat16
    x = xT_ref[0]
    y1 = jnp.dot(w1_ref[...], x, preferred_element_type=F32) + b1_ref[...]
    h1 = _gn_relu(y1, g1_ref[...], be1_ref[...], 16)
    y2 = jnp.dot(w2_ref[...].astype(BF), h1.astype(BF),
                 preferred_element_type=F32) + b2_ref[...]
    h2 = _gn_relu(y2, g2_ref[...], be2_ref[...], 32)
    h2b = h2.astype(BF)
    w3b = w3f_ref[...].astype(BF)

    rsum = jnp.sum(h2, axis=1, keepdims=True)
    s2 = jnp.dot(w3f_ref[...], rsum, preferred_element_type=F32)
    G = lax.dot_general(h2b, h2b, (((1,), (1,)), ((), ())),
                        preferred_element_type=F32)
    M = jnp.dot(w3f_ref[...], G, preferred_element_type=F32)
    q2 = jnp.sum(M * w3f_ref[...], axis=1, keepdims=True)

    NT = 4
    TN = _N // NT
    neg = jnp.float32(-jnp.inf)
    mg = jnp.full((1024, 1), neg, F32)
    mj = jnp.full((1024, 1), neg, F32)
    md = jnp.full((1024, 1), neg, F32)
    mrowj = maskj_ref[0]
    mrowd = maskd_ref[0]
    for t in range(NT):
        sl = slice(t * TN, (t + 1) * TN)
        raw = jnp.dot(w3b, h2b[:, sl], preferred_element_type=F32)
        mg = jnp.maximum(mg, jnp.max(raw, axis=1, keepdims=True))
        selj = jnp.where(mrowj[:, sl] > 0, raw, neg)
        mj = jnp.maximum(mj, jnp.max(selj, axis=1, keepdims=True))
        seld = jnp.where(mrowd[:, sl] > 0, raw, neg)
        md = jnp.maximum(md, jnp.max(seld, axis=1, keepdims=True))

    flip = flip_ref[...]
    b3 = b3_ref[...]
    n = jnp.float32(_N)
    s = flip * s2 + n * b3
    q = q2 + 2.0 * b3 * flip * s2 + n * b3 * b3
    scale, shift = _gn_scale_shift(s, q, g3_ref[...], be3_ref[...], 64, 16 * _N)
    w = scale * flip
    fb3 = flip * b3
    gmax_ref[...] = jnp.maximum(w * (mg + fb3) + shift, 0.0).reshape(1, 1024, 1)
    jm_ref[...] = jnp.maximum(w * (mj + fb3) + shift, 0.0).reshape(1, 1024, 1)
    dm_ref[...] = jnp.maximum(w * (md + fb3) + shift, 0.0).reshape(1, 1024, 1)


def _encode(xT, maskj, maskd, w1, b1, g1, be1, w2, b2, g2, be2,
            w3f, b3, flip, g3, be3):
    col = jax.ShapeDtypeStruct((_B, 1024, 1), F32)
    full = lambda shape: pl.BlockSpec(shape, lambda b: (0,) * len(shape))
    return pl.pallas_call(
        _k2_body,
        grid=(_B,),
        in_specs=[
            pl.BlockSpec((1, 4, _N), lambda b: (b, 0, 0)),
            pl.BlockSpec((1, 1, _N), lambda b: (b, 0, 0)),
            pl.BlockSpec((1, 1, _N), lambda b: (b, 0, 0)),
            full((128, 4)), full((128, 1)), full((128, 1)), full((128, 1)),
            full((256, 128)), full((256, 1)), full((256, 1)), full((256, 1)),
            full((1024, 256)), full((1024, 1)), full((1024, 1)),
            full((1024, 1)), full((1024, 1)),
        ],
        out_specs=[
            pl.BlockSpec((1, 1024, 1), lambda b: (b, 0, 0)),
            pl.BlockSpec((1, 1024, 1), lambda b: (b, 0, 0)),
            pl.BlockSpec((1, 1024, 1), lambda b: (b, 0, 0)),
        ],
        out_shape=[col, col, col],
    )(xT, maskj, maskd, w1, b1, g1, be1, w2, b2, g2, be2,
      w3f, b3, flip, g3, be3)


def _linT(x, w, b):
    return lax.dot_general(x, w, (((1,), (1,)), ((), ())),
                           preferred_element_type=F32) + b


def _ln(x, g, b):
    m = jnp.mean(x, axis=-1, keepdims=True)
    v = jnp.mean(x * x, axis=-1, keepdims=True) - m * m
    return (x - m) * lax.rsqrt(v + _EPS) * g + b


def _relu(x):
    return jnp.maximum(x, 0.0)


def _k3_body(g_ref, jm_ref, dm_ref, dp_ref, dv_ref, jt_ref, ja_ref, jo_ref,
             *prefs):
    names = _K3_PARAM_NAMES
    p = {}
    for nm, r in zip(names, prefs):
        v = r[...]
        p[nm] = v.reshape(1, -1) if v.ndim == 1 else v
    g0 = g_ref[...]
    jm_in = jm_ref[...]
    dm_in = dm_ref[...]
    dp = dp_ref[...]
    dv = dv_ref[...]
    jt = jt_ref[...]
    ja = ja_ref[...]
    jo = jo_ref[...]

    g = _relu(_ln(_linT(g0, p['pc_w4'], p['pc_b4']), p['pc_ln4g'], p['pc_ln4b']))
    g = _linT(g, p['pc_w5'], p['pc_b5'])

    di = jnp.concatenate([dp, dv], axis=1)
    direction_feat = _linT(_ln(_relu(_linT(di, p['de_w1'], p['de_b1'])),
                               p['de_lng'], p['de_lnb']), p['de_w2'], p['de_b2'])
    rel = dp - jo
    relf = _linT(_ln(_relu(_linT(rel, p['rp_w1'], p['rp_b1'])),
                     p['rp_lng'], p['rp_lnb']), p['rp_w2'], p['rp_b2'])
    mag = jnp.sqrt(jnp.sum(dv * dv, axis=1, keepdims=True))
    magf = _linT(_relu(_linT(mag, p['mg_w1'], p['mg_b1'])), p['mg_w2'], p['mg_b2'])
    comb = jnp.concatenate([direction_feat, relf, magf], axis=1)
    drag_feat = _linT(_relu(_linT(comb, p['df_w1'], p['df_b1'])),
                      p['df_w2'], p['df_b2'])

    tf = jnp.where(jt == 0, p['emb'][0:1, :], p['emb'][1:2, :])
    af = _linT(_relu(_linT(ja, p['ax_w1'], p['ax_b1'])), p['ax_w2'], p['ax_b2'])
    of = _linT(_relu(_linT(jo, p['or_w1'], p['or_b1'])), p['or_w2'], p['or_b2'])
    jc = jnp.concatenate([tf, af, of], axis=1)
    joint_feat = _linT(_relu(_linT(jc, p['jf_w1'], p['jf_b1'])),
                       p['jf_w2'], p['jf_b2'])

    jl = _linT(_relu(_linT(jm_in, p['jm_w1'], p['jm_b1'])), p['jm_w2'], p['jm_b2'])
    dl = _linT(_relu(_linT(dm_in, p['dm_w1'], p['dm_b1'])), p['dm_w2'], p['dm_b2'])
    local = _linT(_relu(_linT(jnp.concatenate([jl, dl], axis=1),
                              p['lf_w1'], p['lf_b1'])), p['lf_w2'], p['lf_b2'])
    local = _linT(joint_feat, p['fs_w'], p['fs_b']) * local + \
        _linT(joint_feat, p['fsh_w'], p['fsh_b'])

    vi = jnp.concatenate([local, joint_feat, drag_feat], axis=1)
    mu = _linT(vi, p['mu_w'], p['mu_b'])
    lv = _linT(vi, p['lv_w'], p['lv_b'])
    out_ref = prefs[len(names)]
    out_ref[...] = jnp.concatenate([mu, lv, g], axis=1)


_K3_PARAM_NAMES = (
    'pc_w4', 'pc_b4', 'pc_ln4g', 'pc_ln4b', 'pc_w5', 'pc_b5',
    'de_w1', 'de_b1', 'de_lng', 'de_lnb', 'de_w2', 'de_b2',
    'rp_w1', 'rp_b1', 'rp_lng', 'rp_lnb', 'rp_w2', 'rp_b2',
    'mg_w1', 'mg_b1', 'mg_w2', 'mg_b2',
    'df_w1', 'df_b1', 'df_w2', 'df_b2',
    'emb',
    'ax_w1', 'ax_b1', 'ax_w2', 'ax_b2',
    'or_w1', 'or_b1', 'or_w2', 'or_b2',
    'jf_w1', 'jf_b1', 'jf_w2', 'jf_b2',
    'jm_w1', 'jm_b1', 'jm_w2', 'jm_b2',
    'dm_w1', 'dm_b1', 'dm_w2', 'dm_b2',
    'lf_w1', 'lf_b1', 'lf_w2', 'lf_b2',
    'fs_w', 'fs_b', 'fsh_w', 'fsh_b',
    'mu_w', 'mu_b', 'lv_w', 'lv_b',
)

def _k3_entry(*refs):
    _k3_body(*refs)


def _head(gT, jmT, dmT, dp, dv, jt2, ja, jo, plist):
    return pl.pallas_call(
        _k3_entry,
        out_shape=jax.ShapeDtypeStruct((_B, 2048), F32),
    )(gT, jmT, dmT, dp, dv, jt2, ja, jo, *plist)


def kernel(points, drag_point, drag_vector, joint_type, joint_axis,
           joint_origin, params):
    p = params
    xT = jnp.transpose(points, (0, 2, 1))

    maskj, maskd = _select_masks(xT, joint_origin, drag_point)

    c1 = lambda a: a.reshape(-1, 1)
    flip = jnp.where(p['pc_g3'] >= 0, 1.0, -1.0).astype(F32)
    w3f = p['pc_w3'] * flip[:, None]
    gcols, jmcols, dmcols = _encode(
        xT, maskj, maskd,
        p['pc_w1'], c1(p['pc_b1']), c1(p['pc_g1']), c1(p['pc_be1']),
        p['pc_w2'], c1(p['pc_b2']), c1(p['pc_g2']), c1(p['pc_be2']),
        w3f, c1(p['pc_b3']), c1(flip), c1(p['pc_g3']), c1(p['pc_be3']))

    plist = [p[nm] for nm in _K3_PARAM_NAMES]
    gcols = gcols.reshape(_B, 1024)
    jmcols = jmcols.reshape(_B, 1024)
    dmcols = dmcols.reshape(_B, 1024)
    return _head(gcols, jmcols, dmcols,
                 drag_point, drag_vector,
                 joint_type.astype(jnp.int32).reshape(_B, 1),
                 joint_axis, joint_origin, plist)
